# Initial kernel scaffold; baseline (speedup 1.0000x reference)
import jax
import jax.numpy as jnp
from jax.experimental import pallas as pl


def kernel(agts, ctx, distance, hi, wi, W_dist0, b_dist0, W_dist1, g_dist1, be_dist1, W_q, g_q, be_q, W_ctx0, g_ctx0, be_ctx0, W_ctx1, W_agt, g_norm, be_norm, W_lin, g_lin, be_lin):
    raise NotImplementedError("write your pallas kernel here")


# speedup vs baseline: 1.8690x; 1.8690x over previous
import functools

import jax
import jax.numpy as jnp
from jax import lax
from jax.experimental import pallas as pl
from jax.experimental.pallas import tpu as pltpu
from jax.experimental.pallas import tpu_sc as plsc

NC, NS = 2, 16
NW = NC * NS
CH = 128
_EPS = 1e-5


def _gn(x, gamma, beta):
    m = jnp.mean(x, axis=1, keepdims=True)
    v = jnp.mean((x - m) ** 2, axis=1, keepdims=True)
    return (x - m) / jnp.sqrt(v + _EPS) * gamma + beta


def _dot(a, b):
    return jnp.dot(a, b, preferred_element_type=jnp.float32)


def _node_pre_body(agts_ref, ctx_ref, wq_ref, gq_ref, beq_ref, w0q_ref,
                   w0c_ref, wagt_ref, aq_ref, ac_ref, aagt_ref):
    x = agts_ref[...]
    qn = jnp.maximum(_gn(_dot(x, wq_ref[...]), gq_ref[...], beq_ref[...]), 0.0)
    aq_ref[...] = _dot(qn, w0q_ref[...])
    ac_ref[...] = _dot(ctx_ref[...], w0c_ref[...])
    aagt_ref[...] = _dot(x, wagt_ref[...])


def _edge_body(dist_ref, gq_ref, gc_ref, wd0_ref, bd0_ref, wd1_ref, gd1_ref,
               bed1_ref, w0d_ref, gc0_ref, bec0_ref, wc1_ref, out_ref):
    dist = dist_ref[...]
    w = wd0_ref[...]
    d0 = dist[:, 0:1] * w[0:1, :] + dist[:, 1:2] * w[1:2, :] + bd0_ref[...]
    d0 = jnp.maximum(d0, 0.0)
    t = jnp.maximum(_gn(_dot(d0, wd1_ref[...]), gd1_ref[...], bed1_ref[...]), 0.0)
    s = _dot(t, w0d_ref[...]) + gq_ref[...] + gc_ref[...]
    u = jnp.maximum(_gn(s, gc0_ref[...], bec0_ref[...]), 0.0)
    out_ref[...] = _dot(u, wc1_ref[...])


def _node_post_body(s2_ref, agts_ref, gn_ref, ben_ref, wl_ref, gl_ref,
                    bel_ref, out_ref):
    a = s2_ref[0] + s2_ref[1]
    a = jnp.maximum(_gn(a, gn_ref[...], ben_ref[...]), 0.0)
    a = _gn(_dot(a, wl_ref[...]), gl_ref[...], bel_ref[...])
    out_ref[...] = jnp.maximum(a + agts_ref[...], 0.0)


def _sc_mesh():
    return plsc.VectorSubcoreMesh(core_axis_name="c", subcore_axis_name="s",
                                  num_cores=NC, num_subcores=NS)


def _make_gather(n_tab, d, e_pad):
    cpt = e_pad // NW // CH

    @functools.partial(
        pl.kernel,
        out_type=(jax.ShapeDtypeStruct((e_pad, d), jnp.float32),
                  jax.ShapeDtypeStruct((e_pad, d), jnp.float32)),
        mesh=_sc_mesh(),
        scratch_types=[
            pltpu.VMEM((cpt, CH), jnp.int32),
            pltpu.VMEM((cpt, CH), jnp.int32),
            pltpu.VMEM((2, CH, d), jnp.float32),
            pltpu.VMEM((2, CH, d), jnp.float32),
            pltpu.SemaphoreType.DMA,
            pltpu.SemaphoreType.DMA,
            pltpu.SemaphoreType.DMA,
            pltpu.SemaphoreType.DMA,
        ],
    )
    def gather_k(aq_hbm, ac_hbm, hi2d_hbm, wi2d_hbm, gq_hbm, gc_hbm,
                 hi_v, wi_v, qrows, crows, sq0, sq1, sc0, sc1):
        wid = lax.axis_index("s") * NC + lax.axis_index("c")
        crow0 = wid * cpt
        ebase = crow0 * CH
        pltpu.sync_copy(hi2d_hbm.at[pl.ds(crow0, cpt)], hi_v)
        pltpu.sync_copy(wi2d_hbm.at[pl.ds(crow0, cpt)], wi_v)

        def step(j, carry):
            c0 = j * 2
            cp_q0 = pltpu.async_copy(aq_hbm.at[hi_v.at[c0]], qrows.at[0], sq0)
            cp_c0 = pltpu.async_copy(ac_hbm.at[wi_v.at[c0]], crows.at[0], sc0)
            cp_q1 = pltpu.async_copy(aq_hbm.at[hi_v.at[c0 + 1]], qrows.at[1], sq1)
            cp_c1 = pltpu.async_copy(ac_hbm.at[wi_v.at[c0 + 1]], crows.at[1], sc1)
            cp_q0.wait()
            pltpu.sync_copy(qrows.at[0], gq_hbm.at[pl.ds(ebase + c0 * CH, CH)])
            cp_c0.wait()
            pltpu.sync_copy(crows.at[0], gc_hbm.at[pl.ds(ebase + c0 * CH, CH)])
            cp_q1.wait()
            pltpu.sync_copy(qrows.at[1], gq_hbm.at[pl.ds(ebase + (c0 + 1) * CH, CH)])
            cp_c1.wait()
            pltpu.sync_copy(crows.at[1], gc_hbm.at[pl.ds(ebase + (c0 + 1) * CH, CH)])
            return carry

        lax.fori_loop(0, cpt // 2, step, 0)

    return gather_k


def _make_scatter(d, e_pad, nacc):
    cpt = e_pad // NW // CH
    rpt = nacc // NS

    @functools.partial(
        pl.kernel,
        out_type=jax.ShapeDtypeStruct((NC, nacc, d), jnp.float32),
        mesh=_sc_mesh(),
        scratch_types=[
            pltpu.VMEM((cpt, CH), jnp.int32),
            pltpu.VMEM((2, CH, d), jnp.float32),
            pltpu.VMEM_SHARED((nacc, d), jnp.float32),
            pltpu.SemaphoreType.DMA,
            pltpu.SemaphoreType.DMA,
        ],
    )
    def scatter_k(c_hbm, hi2d_hbm, init_hbm, out_hbm, hi_v, rows, acc, s0, s1):
        core = lax.axis_index("c")
        sid = lax.axis_index("s")
        wid = sid * NC + core
        crow0 = wid * cpt
        ebase = crow0 * CH
        r0 = sid * rpt
        pltpu.sync_copy(init_hbm.at[core].at[pl.ds(r0, rpt)],
                        acc.at[pl.ds(r0, rpt)])
        pltpu.sync_copy(hi2d_hbm.at[pl.ds(crow0, cpt)], hi_v)
        plsc.subcore_barrier()

        def step(j, carry):
            c0 = j * 2
            cp0 = pltpu.async_copy(c_hbm.at[pl.ds(ebase + c0 * CH, CH)],
                                   rows.at[0], s0)
            cp1 = pltpu.async_copy(c_hbm.at[pl.ds(ebase + (c0 + 1) * CH, CH)],
                                   rows.at[1], s1)
            cp0.wait()
            pltpu.sync_copy(rows.at[0], acc.at[hi_v.at[c0]], add=True)
            cp1.wait()
            pltpu.sync_copy(rows.at[1], acc.at[hi_v.at[c0 + 1]], add=True)
            return carry

        lax.fori_loop(0, cpt // 2, step, 0)
        plsc.subcore_barrier()
        pltpu.sync_copy(acc.at[pl.ds(r0, rpt)],
                        out_hbm.at[core].at[pl.ds(r0, rpt)])

    return scatter_k


def kernel(agts, ctx, distance, hi, wi,
           W_dist0, b_dist0, W_dist1, g_dist1, be_dist1,
           W_q, g_q, be_q,
           W_ctx0, g_ctx0, be_ctx0, W_ctx1,
           W_agt, g_norm, be_norm,
           W_lin, g_lin, be_lin):
    N, D = agts.shape
    E = hi.shape[0]
    R = 512
    e_pad = -(-E // (NW * CH)) * (NW * CH)
    nacc = -(-N // R) * R

    pad = e_pad - E
    hi_p = jnp.concatenate([hi, jnp.full((pad,), N, jnp.int32)])
    wi_p = jnp.concatenate([wi, jnp.zeros((pad,), jnp.int32)])
    dist_p = jnp.concatenate([distance, jnp.zeros((pad, 2), jnp.float32)])
    hi2d = hi_p.reshape(e_pad // CH, CH)
    wi2d = wi_p.reshape(e_pad // CH, CH)

    row = lambda v: v.reshape(1, D)
    w0dT = W_ctx0[:, :D].T
    w0qT = W_ctx0[:, D:2 * D].T
    w0cT = W_ctx0[:, 2 * D:].T

    full = pl.BlockSpec((D, D), lambda i: (0, 0))
    vec = pl.BlockSpec((1, D), lambda i: (0, 0))
    rblk = pl.BlockSpec((R, D), lambda i: (i, 0))

    aq, ac, aagt = pl.pallas_call(
        _node_pre_body,
        grid=(nacc // R,),
        in_specs=[rblk, rblk, full, vec, vec, full, full, full],
        out_specs=[rblk, rblk, rblk],
        out_shape=[jax.ShapeDtypeStruct((nacc, D), jnp.float32)] * 3,
    )(agts, ctx, W_q.T, row(g_q), row(be_q), w0qT, w0cT, W_agt.T)

    gq, gc = _make_gather(nacc, D, e_pad)(aq, ac, hi2d, wi2d)

    msgs = pl.pallas_call(
        _edge_body,
        grid=(e_pad // R,),
        in_specs=[pl.BlockSpec((R, 2), lambda i: (i, 0)), rblk, rblk,
                  pl.BlockSpec((2, D), lambda i: (0, 0)), vec, full, vec,
                  vec, full, vec, vec, full],
        out_specs=rblk,
        out_shape=jax.ShapeDtypeStruct((e_pad, D), jnp.float32),
    )(dist_p, gq, gc, W_dist0.T, row(b_dist0), W_dist1.T, row(g_dist1),
      row(be_dist1), w0dT, row(g_ctx0), row(be_ctx0), W_ctx1.T)

    init = jnp.stack([aagt, jnp.zeros_like(aagt)])
    partials = _make_scatter(D, e_pad, nacc)(msgs, hi2d, init)

    out = pl.pallas_call(
        _node_post_body,
        grid=(nacc // R,),
        in_specs=[pl.BlockSpec((NC, R, D), lambda i: (0, i, 0)), rblk, vec,
                  vec, full, vec, vec],
        out_specs=rblk,
        out_shape=jax.ShapeDtypeStruct((N, D), jnp.float32),
    )(partials, agts, row(g_norm), row(be_norm), W_lin.T, row(g_lin),
      row(be_lin))
    return out


# speedup vs baseline: 2.5999x; 1.3911x over previous
import functools

import jax
import jax.numpy as jnp
from jax import lax
from jax.experimental import pallas as pl
from jax.experimental.pallas import tpu as pltpu
from jax.experimental.pallas import tpu_sc as plsc

NC, NS = 2, 16
NW = NC * NS
CH = 128
_EPS = 1e-5


def _gn(x, gamma, beta):
    m = jnp.mean(x, axis=1, keepdims=True)
    v = jnp.mean((x - m) ** 2, axis=1, keepdims=True)
    return (x - m) / jnp.sqrt(v + _EPS) * gamma + beta


def _dot(a, b):
    return jnp.dot(a, b, preferred_element_type=jnp.float32)


def _node_pre_body(agts_ref, ctx_ref, wq_ref, gq_ref, beq_ref, w0q_ref,
                   w0c_ref, wagt_ref, tabs_ref, aagt_ref):
    x = agts_ref[...]
    qn = jnp.maximum(_gn(_dot(x, wq_ref[...]), gq_ref[...], beq_ref[...]), 0.0)
    tabs_ref[0] = _dot(qn, w0q_ref[...])
    tabs_ref[1] = _dot(ctx_ref[...], w0c_ref[...])
    aagt_ref[...] = _dot(x, wagt_ref[...])


def _edge_body(dist_ref, g2_ref, wd0_ref, bd0_ref, wd1_ref, gd1_ref,
               bed1_ref, w0d_ref, gc0_ref, bec0_ref, wc1_ref, out_ref):
    dist = dist_ref[...]
    w = wd0_ref[...]
    d0 = dist[:, 0:1] * w[0:1, :] + dist[:, 1:2] * w[1:2, :] + bd0_ref[...]
    d0 = jnp.maximum(d0, 0.0)
    t = jnp.maximum(_gn(_dot(d0, wd1_ref[...]), gd1_ref[...], bed1_ref[...]), 0.0)
    s = _dot(t, w0d_ref[...]) + g2_ref[0] + g2_ref[1]
    u = jnp.maximum(_gn(s, gc0_ref[...], bec0_ref[...]), 0.0)
    out_ref[...] = _dot(u, wc1_ref[...])


def _node_post_body(s2_ref, agts_ref, gn_ref, ben_ref, wl_ref, gl_ref,
                    bel_ref, out_ref):
    a = s2_ref[0] + s2_ref[1]
    a = jnp.maximum(_gn(a, gn_ref[...], ben_ref[...]), 0.0)
    a = _gn(_dot(a, wl_ref[...]), gl_ref[...], bel_ref[...])
    out_ref[...] = jnp.maximum(a + agts_ref[...], 0.0)


def _sc_mesh():
    return plsc.VectorSubcoreMesh(core_axis_name="c", subcore_axis_name="s",
                                  num_cores=NC, num_subcores=NS)


def _make_gather(nacc, d, e_pad):
    cpt = e_pad // NS // CH
    rpt = nacc // NS

    @functools.partial(
        pl.kernel,
        out_type=jax.ShapeDtypeStruct((NC, e_pad, d), jnp.float32),
        mesh=_sc_mesh(),
        scratch_types=[
            pltpu.VMEM((cpt, CH), jnp.int32),
            pltpu.VMEM((2, CH, d), jnp.float32),
            pltpu.VMEM_SHARED((nacc, d), jnp.float32),
            pltpu.SemaphoreType.DMA,
            pltpu.SemaphoreType.DMA,
        ],
    )
    def gather_k(tabs_hbm, idx3d_hbm, g2_hbm, idx_v, rows, tab, s0, s1):
        core = lax.axis_index("c")
        sid = lax.axis_index("s")
        r0 = sid * rpt
        crow0 = sid * cpt
        ebase = crow0 * CH
        pltpu.sync_copy(tabs_hbm.at[core].at[pl.ds(r0, rpt)],
                        tab.at[pl.ds(r0, rpt)])
        pltpu.sync_copy(idx3d_hbm.at[core].at[pl.ds(crow0, cpt)], idx_v)
        plsc.subcore_barrier()

        def step(j, carry):
            c0 = j * 2
            cp0 = pltpu.async_copy(tab.at[idx_v.at[c0]], rows.at[0], s0)
            cp1 = pltpu.async_copy(tab.at[idx_v.at[c0 + 1]], rows.at[1], s1)
            cp0.wait()
            pltpu.sync_copy(rows.at[0],
                            g2_hbm.at[core].at[pl.ds(ebase + c0 * CH, CH)])
            cp1.wait()
            pltpu.sync_copy(rows.at[1],
                            g2_hbm.at[core].at[pl.ds(ebase + (c0 + 1) * CH, CH)])
            return carry

        lax.fori_loop(0, cpt // 2, step, 0)

    return gather_k


def _make_scatter(d, e_pad, nacc):
    cpt = e_pad // NW // CH
    rpt = nacc // NS

    @functools.partial(
        pl.kernel,
        out_type=jax.ShapeDtypeStruct((NC, nacc, d), jnp.float32),
        mesh=_sc_mesh(),
        scratch_types=[
            pltpu.VMEM((cpt, CH), jnp.int32),
            pltpu.VMEM((2, CH, d), jnp.float32),
            pltpu.VMEM_SHARED((nacc, d), jnp.float32),
            pltpu.SemaphoreType.DMA,
            pltpu.SemaphoreType.DMA,
        ],
    )
    def scatter_k(c_hbm, hi2d_hbm, init_hbm, out_hbm, hi_v, rows, acc, s0, s1):
        core = lax.axis_index("c")
        sid = lax.axis_index("s")
        wid = sid * NC + core
        crow0 = wid * cpt
        ebase = crow0 * CH
        r0 = sid * rpt
        pltpu.sync_copy(init_hbm.at[core].at[pl.ds(r0, rpt)],
                        acc.at[pl.ds(r0, rpt)])
        pltpu.sync_copy(hi2d_hbm.at[pl.ds(crow0, cpt)], hi_v)
        plsc.subcore_barrier()

        def step(j, carry):
            c0 = j * 2
            cp0 = pltpu.async_copy(c_hbm.at[pl.ds(ebase + c0 * CH, CH)],
                                   rows.at[0], s0)
            cp1 = pltpu.async_copy(c_hbm.at[pl.ds(ebase + (c0 + 1) * CH, CH)],
                                   rows.at[1], s1)
            cp0.wait()
            pltpu.sync_copy(rows.at[0], acc.at[hi_v.at[c0]], add=True)
            cp1.wait()
            pltpu.sync_copy(rows.at[1], acc.at[hi_v.at[c0 + 1]], add=True)
            return carry

        lax.fori_loop(0, cpt // 2, step, 0)
        plsc.subcore_barrier()
        pltpu.sync_copy(acc.at[pl.ds(r0, rpt)],
                        out_hbm.at[core].at[pl.ds(r0, rpt)])

    return scatter_k


def kernel(agts, ctx, distance, hi, wi,
           W_dist0, b_dist0, W_dist1, g_dist1, be_dist1,
           W_q, g_q, be_q,
           W_ctx0, g_ctx0, be_ctx0, W_ctx1,
           W_agt, g_norm, be_norm,
           W_lin, g_lin, be_lin):
    N, D = agts.shape
    E = hi.shape[0]
    R = 512
    e_pad = -(-E // (NW * CH)) * (NW * CH)
    nacc = -(-N // R) * R

    pad = e_pad - E
    hi_p = jnp.concatenate([hi, jnp.full((pad,), N, jnp.int32)])
    wi_p = jnp.concatenate([wi, jnp.zeros((pad,), jnp.int32)])
    dist_p = jnp.concatenate([distance, jnp.zeros((pad, 2), jnp.float32)])
    hi2d = hi_p.reshape(e_pad // CH, CH)
    wi2d = wi_p.reshape(e_pad // CH, CH)
    idx3d = jnp.stack([hi2d, wi2d])

    row = lambda v: v.reshape(1, D)
    w0dT = W_ctx0[:, :D].T
    w0qT = W_ctx0[:, D:2 * D].T
    w0cT = W_ctx0[:, 2 * D:].T

    full = pl.BlockSpec((D, D), lambda i: (0, 0))
    vec = pl.BlockSpec((1, D), lambda i: (0, 0))
    rblk = pl.BlockSpec((R, D), lambda i: (i, 0))

    tabs, aagt = pl.pallas_call(
        _node_pre_body,
        grid=(nacc // R,),
        in_specs=[rblk, rblk, full, vec, vec, full, full, full],
        out_specs=[pl.BlockSpec((NC, R, D), lambda i: (0, i, 0)), rblk],
        out_shape=[jax.ShapeDtypeStruct((NC, nacc, D), jnp.float32),
                   jax.ShapeDtypeStruct((nacc, D), jnp.float32)],
    )(agts, ctx, W_q.T, row(g_q), row(be_q), w0qT, w0cT, W_agt.T)

    g2 = _make_gather(nacc, D, e_pad)(tabs, idx3d)

    msgs = pl.pallas_call(
        _edge_body,
        grid=(e_pad // R,),
        in_specs=[pl.BlockSpec((R, 2), lambda i: (i, 0)),
                  pl.BlockSpec((NC, R, D), lambda i: (0, i, 0)),
                  pl.BlockSpec((2, D), lambda i: (0, 0)), vec, full, vec,
                  vec, full, vec, vec, full],
        out_specs=rblk,
        out_shape=jax.ShapeDtypeStruct((e_pad, D), jnp.float32),
    )(dist_p, g2, W_dist0.T, row(b_dist0), W_dist1.T, row(g_dist1),
      row(be_dist1), w0dT, row(g_ctx0), row(be_ctx0), W_ctx1.T)

    init = jnp.stack([aagt, jnp.zeros_like(aagt)])
    partials = _make_scatter(D, e_pad, nacc)(msgs, hi2d, init)

    out = pl.pallas_call(
        _node_post_body,
        grid=(nacc // R,),
        in_specs=[pl.BlockSpec((NC, R, D), lambda i: (0, i, 0)), rblk, vec,
                  vec, full, vec, vec],
        out_specs=rblk,
        out_shape=jax.ShapeDtypeStruct((N, D), jnp.float32),
    )(partials, agts, row(g_norm), row(be_norm), W_lin.T, row(g_lin),
      row(be_lin))
    return out
